# baseline (speedup 1.0000x reference)
import functools

import jax
import jax.numpy as jnp
from jax.experimental import pallas as pl
from jax.experimental.pallas import tpu as pltpu

_C = 32
_K = 4


def _box3(x, hh, ww):
    xr = x[:, 0:ww] + x[:, 1:ww + 1] + x[:, 2:ww + 2]
    return xr[0:hh] + xr[1:hh + 1] + xr[2:hh + 2]


def _unfold_from(xp, hh, ww):
    return jnp.concatenate(
        [xp[i:i + hh, j:j + ww, :] for i in range(3) for j in range(3)],
        axis=-1)


def _agg_unit(a_blk, b_blk, d, hh, ww, y0, h_total, agg_w_ref, agg_b_ref,
              proj_w_t, proj_b_row):
    c = a_blk.shape[-1]
    r = d // 2
    d2 = d * d
    asq = jnp.sum(a_blk * a_blk, axis=-1)
    an = jnp.sqrt(_box3(asq, hh, ww))
    arn = 1.0 / jnp.maximum(an, 1e-12)
    bsq = jnp.sum(b_blk * b_blk, axis=-1)
    bn = jnp.sqrt(_box3(bsq, hh + 2 * r, ww + 2 * r))
    iy = jax.lax.broadcasted_iota(jnp.int32, (hh + 2 * r, ww + 2 * r), 0)
    ix = jax.lax.broadcasted_iota(jnp.int32, (hh + 2 * r, ww + 2 * r), 1)
    iy = iy + (y0 - r)
    valid = ((iy >= 0) & (iy < h_total) & (ix >= r) & (ix < ww + r))
    brn = valid.astype(jnp.float32) / jnp.maximum(bn, 1e-12)
    buh = _unfold_from(b_blk, hh + 2 * r, ww + 2 * r)
    buh = buh * valid.astype(jnp.float32)[:, :, None]
    sims = [_box3(jnp.sum(a_blk * b_blk[oi:oi + hh + 2, oj:oj + ww + 2, :],
                          axis=-1), hh, ww)
            * (arn * brn[oi:oi + hh, oj:oj + ww])
            for oi in range(d) for oj in range(d)]
    wmap = [jnp.zeros((hh, ww), jnp.float32) for _ in range(d2)]
    for k in range(_K):
        best_v = jnp.full((hh, ww), -jnp.inf, jnp.float32)
        best_i = jnp.zeros((hh, ww), jnp.int32)
        for dd in range(d2):
            m = sims[dd] > best_v
            best_v = jnp.where(m, sims[dd], best_v)
            best_i = jnp.where(m, dd, best_i)
        ak = agg_w_ref[k].astype(jnp.bfloat16).astype(jnp.float32)
        for dd in range(d2):
            hit = best_i == dd
            wmap[dd] = wmap[dd] + ak * hit.astype(jnp.float32)
            sims[dd] = jnp.where(hit, -jnp.inf, sims[dd])
    buh_b = buh.astype(jnp.bfloat16).astype(jnp.float32)
    agg = jnp.zeros((hh, ww, 9 * c), jnp.float32)
    for dd in range(d2):
        oi, oj = dd // d, dd % d
        agg = agg + wmap[dd][:, :, None] * buh_b[oi:oi + hh, oj:oj + ww, :]
    agg = agg + agg_b_ref[0]
    cat = jnp.concatenate([a_blk[1:1 + hh, 1:1 + ww, :],
                           b_blk[r + 1:r + 1 + hh, r + 1:r + 1 + ww, :]],
                          axis=-1).reshape(hh * ww, 2 * c)
    wp = jnp.dot(cat, proj_w_t, preferred_element_type=jnp.float32)
    wp = (wp + proj_b_row).reshape(hh, ww, 9 * c)
    out = agg[:, :, 0:c] * wp[:, :, 0:c]
    for pp in range(1, 9):
        sl = slice(pp * c, (pp + 1) * c)
        out = out + agg[:, :, sl] * wp[:, :, sl]
    return out


def _up2x_quad(x):
    top = jnp.concatenate([x[:1], x[:-1]], axis=0)
    bot = jnp.concatenate([x[1:], x[-1:]], axis=0)
    ya = 0.25 * top + 0.75 * x
    yb = 0.75 * x + 0.25 * bot
    outs = []
    for y in (ya, yb):
        lf = jnp.concatenate([y[:, :1], y[:, :-1]], axis=1)
        rt = jnp.concatenate([y[:, 1:], y[:, -1:]], axis=1)
        outs.append(0.25 * lf + 0.75 * y)
        outs.append(0.75 * y + 0.25 * rt)
    return jnp.concatenate([o[None] for o in outs], axis=0)


def _level_body(ftp_ref, attnp_ref, agg_w_ref, agg_b_ref,
                proj_wt_ref, proj_b_ref, out_ref, *, d, ty, w, h, upsample):
    r = d // 2
    m = r + 1
    t = pl.program_id(0)
    y0 = pl.program_id(1) * ty
    a_rows = pl.ds(y0 + m - 1, ty + 2)
    acf = ftp_ref[1, a_rows, m - 1:m + w + 1, :]
    aat = attnp_ref[t, a_rows, m - 1:m + w + 1, :]
    b_rows = pl.ds(y0, ty + 2 * r + 2)
    bft = ftp_ref[t, b_rows, :, :]
    bat = attnp_ref[t, b_rows, :, :]
    a_blk = acf + aat
    b_blk = bft + bat
    o = _agg_unit(a_blk, b_blk, d, ty, w, y0, h, agg_w_ref, agg_b_ref,
                  proj_wt_ref[...], proj_b_ref[...])
    if upsample:
        out_ref[0] = _up2x_quad(o)
    else:
        out_ref[0] = o


def _level_call(ft, attn, agg_w, agg_b, proj_w_t, proj_b_row, d, ty,
                upsample):
    _, h, w, c = ft.shape
    r = d // 2
    m = r + 1
    pad = ((0, 0), (m, m), (m, m), (0, 0))
    ftp = jnp.pad(ft, pad)
    attnp = jnp.pad(attn, pad)
    body = functools.partial(_level_body, d=d, ty=ty, w=w, h=h,
                             upsample=upsample)
    vm = pl.BlockSpec(memory_space=pltpu.VMEM)
    sm = pl.BlockSpec(memory_space=pltpu.SMEM)
    if upsample:
        oshape = (3, 4, h, w, c)
        ospec = pl.BlockSpec((1, 4, ty, w, c), lambda t, i: (t, 0, i, 0, 0))
    else:
        oshape = (3, h, w, c)
        ospec = pl.BlockSpec((1, ty, w, c), lambda t, i: (t, i, 0, 0))
    return pl.pallas_call(
        body,
        grid=(3, h // ty),
        out_shape=jax.ShapeDtypeStruct(oshape, jnp.float32),
        in_specs=[vm, vm, sm, sm, vm, vm],
        out_specs=ospec,
    )(ftp, attnp, agg_w, agg_b, proj_w_t, proj_b_row)


def _upconv_body(o0_ref, o1_ref, o2_ref, up_wt_ref, up_b_ref, out_ref):
    h, w, c = o0_ref.shape
    cat = jnp.concatenate([o0_ref[...], o1_ref[...], o2_ref[...]], axis=-1)
    up = jnp.dot(cat.reshape(h * w, 3 * c), up_wt_ref[...],
                 preferred_element_type=jnp.float32)
    up = up + up_b_ref[...]
    out_ref[...] = up.reshape(h, w, 4 * _C)


def _upconv_call(outs, up_w_t, up_b_row, h, w):
    vm = pl.BlockSpec(memory_space=pltpu.VMEM)
    return pl.pallas_call(
        _upconv_body,
        out_shape=jax.ShapeDtypeStruct((h, w, 4 * _C), jnp.float32),
        in_specs=[vm, vm, vm, vm, vm],
        out_specs=vm,
    )(outs[0], outs[1], outs[2], up_w_t, up_b_row)


def _interleave_up(q):
    t, _, h, w, c = q.shape
    q = q.reshape(t, 2, 2, h, w, c)
    q = jnp.transpose(q, (0, 3, 1, 4, 2, 5))
    return q.reshape(t, 2 * h, 2 * w, c)


def _perm_proj(proj_w, proj_b):
    wt = proj_w.reshape(_C, 9, 2 * _C).transpose(1, 0, 2).reshape(9 * _C, 2 * _C)
    return wt.T, proj_b.reshape(_C, 9).T.reshape(1, 9 * _C)


def kernel(feats_l1, feats_l2, feats_l3, au3_agg_w, au3_agg_b, au3_proj_w,
           au3_proj_b, au2_agg_w, au2_agg_b, au2_proj_w, au2_proj_b,
           au1_agg_w, au1_agg_b, au1_proj_w, au1_proj_b, up_w, up_b):
    f1 = jnp.transpose(feats_l1[0], (1, 2, 3, 0))
    f2 = jnp.transpose(feats_l2[0], (1, 2, 3, 0))
    f3 = jnp.transpose(feats_l3[0], (1, 2, 3, 0))
    p3wt, p3b = _perm_proj(au3_proj_w, au3_proj_b)
    p2wt, p2b = _perm_proj(au2_proj_w, au2_proj_b)
    p1wt, p1b = _perm_proj(au1_proj_w, au1_proj_b)
    z3 = jnp.zeros((3, 10, 10, _C), jnp.float32)
    q3 = _level_call(f3, z3, au3_agg_w, au3_agg_b, p3wt, p3b, 3, 10, True)
    a3 = _interleave_up(q3)
    q2 = _level_call(f2, a3, au2_agg_w, au2_agg_b, p2wt, p2b, 5, 20, True)
    a2 = _interleave_up(q2)
    o1t = _level_call(f1, a2, au1_agg_w, au1_agg_b, p1wt, p1b, 7, 10, False)
    o1 = _upconv_call([o1t[0], o1t[1], o1t[2]], up_w.T,
                      up_b.reshape(1, 4 * _C), 40, 40)
    o1 = o1.reshape(40, 40, _C, 2, 2)
    o1 = jnp.transpose(o1, (2, 0, 3, 1, 4))
    return o1.reshape(1, _C, 80, 80)


# speedup vs baseline: 1.0560x; 1.0560x over previous
import functools

import jax
import jax.numpy as jnp
from jax.experimental import pallas as pl
from jax.experimental.pallas import tpu as pltpu

_C = 32
_K = 4


def _unfold_from(xp, hh, ww):
    return jnp.concatenate(
        [xp[i:i + hh, j:j + ww, :] for i in range(3) for j in range(3)],
        axis=-1)


def _agg_unit(a_blk, b_blk, d, hh, ww, y0, h_total, agg_w_ref, agg_b_ref,
              proj_w_t, proj_b_row):
    c = a_blk.shape[-1]
    r = d // 2
    d2 = d * d
    au = _unfold_from(a_blk, hh, ww)
    an = jnp.sqrt(jnp.sum(au * au, axis=-1, keepdims=True))
    ahat = au / jnp.maximum(an, 1e-12)
    buh = _unfold_from(b_blk, hh + 2 * r, ww + 2 * r)
    iy = jax.lax.broadcasted_iota(jnp.int32, (hh + 2 * r, ww + 2 * r), 0)
    ix = jax.lax.broadcasted_iota(jnp.int32, (hh + 2 * r, ww + 2 * r), 1)
    iy = iy + (y0 - r)
    valid = ((iy >= 0) & (iy < h_total) & (ix >= r) & (ix < ww + r))
    buh = buh * valid.astype(jnp.float32)[:, :, None]
    bn = jnp.sqrt(jnp.sum(buh * buh, axis=-1))
    brn = 1.0 / jnp.maximum(bn, 1e-12)
    sims = [jnp.sum(ahat * buh[oi:oi + hh, oj:oj + ww, :], axis=-1)
            * brn[oi:oi + hh, oj:oj + ww]
            for oi in range(d) for oj in range(d)]
    wmap = [jnp.zeros((hh, ww), jnp.float32) for _ in range(d2)]
    for k in range(_K):
        best_v = jnp.full((hh, ww), -jnp.inf, jnp.float32)
        best_i = jnp.zeros((hh, ww), jnp.int32)
        for dd in range(d2):
            m = sims[dd] > best_v
            best_v = jnp.where(m, sims[dd], best_v)
            best_i = jnp.where(m, dd, best_i)
        ak = agg_w_ref[k].astype(jnp.bfloat16).astype(jnp.float32)
        for dd in range(d2):
            hit = best_i == dd
            wmap[dd] = wmap[dd] + ak * hit.astype(jnp.float32)
            sims[dd] = jnp.where(hit, -jnp.inf, sims[dd])
    buh_b = buh.astype(jnp.bfloat16).astype(jnp.float32)
    agg = jnp.zeros((hh, ww, 9 * c), jnp.float32)
    for dd in range(d2):
        oi, oj = dd // d, dd % d
        agg = agg + wmap[dd][:, :, None] * buh_b[oi:oi + hh, oj:oj + ww, :]
    agg = agg + agg_b_ref[0]
    cat = jnp.concatenate([a_blk[1:1 + hh, 1:1 + ww, :],
                           b_blk[r + 1:r + 1 + hh, r + 1:r + 1 + ww, :]],
                          axis=-1).reshape(hh * ww, 2 * c)
    wp = jnp.dot(cat, proj_w_t, preferred_element_type=jnp.float32)
    wp = (wp + proj_b_row).reshape(hh, ww, 9 * c)
    out = agg[:, :, 0:c] * wp[:, :, 0:c]
    for pp in range(1, 9):
        sl = slice(pp * c, (pp + 1) * c)
        out = out + agg[:, :, sl] * wp[:, :, sl]
    return out


def _up2x_quad(x):
    top = jnp.concatenate([x[:1], x[:-1]], axis=0)
    bot = jnp.concatenate([x[1:], x[-1:]], axis=0)
    ya = 0.25 * top + 0.75 * x
    yb = 0.75 * x + 0.25 * bot
    outs = []
    for y in (ya, yb):
        lf = jnp.concatenate([y[:, :1], y[:, :-1]], axis=1)
        rt = jnp.concatenate([y[:, 1:], y[:, -1:]], axis=1)
        outs.append(0.25 * lf + 0.75 * y)
        outs.append(0.75 * y + 0.25 * rt)
    return jnp.concatenate([o[None] for o in outs], axis=0)


def _level_body(ftp_ref, attnp_ref, agg_w_ref, agg_b_ref,
                proj_wt_ref, proj_b_ref, out_ref, *, d, ty, w, h, upsample):
    r = d // 2
    m = r + 1
    t = pl.program_id(0)
    y0 = pl.program_id(1) * ty
    a_rows = pl.ds(y0 + m - 1, ty + 2)
    acf = ftp_ref[1, a_rows, m - 1:m + w + 1, :]
    aat = attnp_ref[t, a_rows, m - 1:m + w + 1, :]
    b_rows = pl.ds(y0, ty + 2 * r + 2)
    bft = ftp_ref[t, b_rows, :, :]
    bat = attnp_ref[t, b_rows, :, :]
    a_blk = acf + aat
    b_blk = bft + bat
    o = _agg_unit(a_blk, b_blk, d, ty, w, y0, h, agg_w_ref, agg_b_ref,
                  proj_wt_ref[...], proj_b_ref[...])
    if upsample:
        out_ref[0] = _up2x_quad(o)
    else:
        out_ref[0] = o


def _level_call(ft, attn, agg_w, agg_b, proj_w_t, proj_b_row, d, ty,
                upsample):
    _, h, w, c = ft.shape
    r = d // 2
    m = r + 1
    pad = ((0, 0), (m, m), (m, m), (0, 0))
    ftp = jnp.pad(ft, pad)
    attnp = jnp.pad(attn, pad)
    body = functools.partial(_level_body, d=d, ty=ty, w=w, h=h,
                             upsample=upsample)
    vm = pl.BlockSpec(memory_space=pltpu.VMEM)
    sm = pl.BlockSpec(memory_space=pltpu.SMEM)
    if upsample:
        oshape = (3, 4, h, w, c)
        ospec = pl.BlockSpec((1, 4, ty, w, c), lambda t, i: (t, 0, i, 0, 0))
    else:
        oshape = (3, h, w, c)
        ospec = pl.BlockSpec((1, ty, w, c), lambda t, i: (t, i, 0, 0))
    return pl.pallas_call(
        body,
        grid=(3, h // ty),
        out_shape=jax.ShapeDtypeStruct(oshape, jnp.float32),
        in_specs=[vm, vm, sm, sm, vm, vm],
        out_specs=ospec,
    )(ftp, attnp, agg_w, agg_b, proj_w_t, proj_b_row)


def _upconv_body(o0_ref, o1_ref, o2_ref, up_wt_ref, up_b_ref, out_ref):
    h, w, c = o0_ref.shape
    cat = jnp.concatenate([o0_ref[...], o1_ref[...], o2_ref[...]], axis=-1)
    up = jnp.dot(cat.reshape(h * w, 3 * c), up_wt_ref[...],
                 preferred_element_type=jnp.float32)
    up = up + up_b_ref[...]
    out_ref[...] = up.reshape(h, w, 4 * _C)


def _upconv_call(outs, up_w_t, up_b_row, h, w):
    vm = pl.BlockSpec(memory_space=pltpu.VMEM)
    return pl.pallas_call(
        _upconv_body,
        out_shape=jax.ShapeDtypeStruct((h, w, 4 * _C), jnp.float32),
        in_specs=[vm, vm, vm, vm, vm],
        out_specs=vm,
    )(outs[0], outs[1], outs[2], up_w_t, up_b_row)


def _interleave_up(q):
    t, _, h, w, c = q.shape
    q = q.reshape(t, 2, 2, h, w, c)
    q = jnp.transpose(q, (0, 3, 1, 4, 2, 5))
    return q.reshape(t, 2 * h, 2 * w, c)


def _perm_proj(proj_w, proj_b):
    wt = proj_w.reshape(_C, 9, 2 * _C).transpose(1, 0, 2).reshape(9 * _C, 2 * _C)
    return wt.T, proj_b.reshape(_C, 9).T.reshape(1, 9 * _C)


def kernel(feats_l1, feats_l2, feats_l3, au3_agg_w, au3_agg_b, au3_proj_w,
           au3_proj_b, au2_agg_w, au2_agg_b, au2_proj_w, au2_proj_b,
           au1_agg_w, au1_agg_b, au1_proj_w, au1_proj_b, up_w, up_b):
    f1 = jnp.transpose(feats_l1[0], (1, 2, 3, 0))
    f2 = jnp.transpose(feats_l2[0], (1, 2, 3, 0))
    f3 = jnp.transpose(feats_l3[0], (1, 2, 3, 0))
    p3wt, p3b = _perm_proj(au3_proj_w, au3_proj_b)
    p2wt, p2b = _perm_proj(au2_proj_w, au2_proj_b)
    p1wt, p1b = _perm_proj(au1_proj_w, au1_proj_b)
    z3 = jnp.zeros((3, 10, 10, _C), jnp.float32)
    q3 = _level_call(f3, z3, au3_agg_w, au3_agg_b, p3wt, p3b, 3, 10, True)
    a3 = _interleave_up(q3)
    q2 = _level_call(f2, a3, au2_agg_w, au2_agg_b, p2wt, p2b, 5, 20, True)
    a2 = _interleave_up(q2)
    o1t = _level_call(f1, a2, au1_agg_w, au1_agg_b, p1wt, p1b, 7, 10, False)
    o1 = _upconv_call([o1t[0], o1t[1], o1t[2]], up_w.T,
                      up_b.reshape(1, 4 * _C), 40, 40)
    o1 = o1.reshape(40, 40, _C, 2, 2)
    o1 = jnp.transpose(o1, (2, 0, 3, 1, 4))
    return o1.reshape(1, _C, 80, 80)
